# Initial kernel scaffold; baseline (speedup 1.0000x reference)
import functools

import jax
import jax.numpy as jnp
from jax import lax
from jax.experimental import pallas as pl
from jax.experimental.pallas import tpu as pltpu
from jax.experimental.pallas import tpu_sc as plsc

N_NODES = 10000
N_EDGES = 320000
HIDDEN = 128
LANES = 16

CHUNK = 128
NCHUNKS = N_EDGES // CHUNK
NTILES = 32
ITERS = -(-NCHUNKS // NTILES)
ROWS_PER_TILE = N_NODES // 16

_mesh = plsc.VectorSubcoreMesh(core_axis_name="c", subcore_axis_name="s")


@functools.partial(
    pl.kernel,
    out_type=jax.ShapeDtypeStruct((2, N_NODES, HIDDEN), jnp.float32),
    mesh=_mesh,
    scratch_types=[
        pltpu.VMEM((CHUNK,), jnp.int32),
        pltpu.VMEM((CHUNK,), jnp.int32),
        pltpu.VMEM((CHUNK,), jnp.float32),
        pltpu.VMEM((CHUNK, HIDDEN), jnp.float32),
        pltpu.VMEM_SHARED((N_NODES, HIDDEN), jnp.float32),
        pltpu.SemaphoreType.DMA,
    ],
)
def _segment_sum_sc(x_hbm, src_hbm, dst_hbm, a_hbm, zeros_hbm, out_hbm,
                    src_v, dst_v, a_v, rows_v, agg_sh, sem):
    cid = lax.axis_index("c")
    sid = lax.axis_index("s")
    wid = sid * 2 + cid
    row0 = sid * ROWS_PER_TILE

    pltpu.sync_copy(zeros_hbm, agg_sh.at[pl.ds(row0, ROWS_PER_TILE)])
    plsc.subcore_barrier()

    def chunk_body(i, carry):
        c = wid + NTILES * i

        @pl.when(c < NCHUNKS)
        def _():
            base = c * CHUNK
            pltpu.sync_copy(src_hbm.at[pl.ds(base, CHUNK)], src_v)
            pltpu.sync_copy(dst_hbm.at[pl.ds(base, CHUNK)], dst_v)
            pltpu.sync_copy(a_hbm.at[pl.ds(base, CHUNK)], a_v)
            pltpu.async_copy(x_hbm.at[src_v], rows_v, sem).wait()

            def edge_body(e, carry2):
                a16 = plsc.load_gather(a_v, [jnp.full((LANES,), e, jnp.int32)])
                for j in range(HIDDEN // LANES):
                    sl = pl.ds(j * LANES, LANES)
                    rows_v[e, sl] = rows_v[e, sl] * a16
                return carry2

            lax.fori_loop(0, CHUNK, edge_body, 0)
            pltpu.sync_copy(rows_v, agg_sh.at[dst_v], add=True)

        return carry

    lax.fori_loop(0, ITERS, chunk_body, 0)

    plsc.subcore_barrier()
    pltpu.sync_copy(agg_sh.at[pl.ds(row0, ROWS_PER_TILE)],
                    out_hbm.at[cid, pl.ds(row0, ROWS_PER_TILE)])


ROW_BLOCK = 1000


def _linear_relu_body(p0_ref, p1_ref, w_ref, b_ref, o_ref):
    s = p0_ref[...] + p1_ref[...]
    y = lax.dot_general(s, w_ref[...], (((1,), (1,)), ((), ())),
                        preferred_element_type=jnp.float32)
    o_ref[...] = jnp.maximum(y + b_ref[...], 0.0)


def _linear_relu(p0, p1, W, b2d):
    return pl.pallas_call(
        _linear_relu_body,
        grid=(N_NODES // ROW_BLOCK,),
        in_specs=[
            pl.BlockSpec((ROW_BLOCK, HIDDEN), lambda i: (i, 0)),
            pl.BlockSpec((ROW_BLOCK, HIDDEN), lambda i: (i, 0)),
            pl.BlockSpec((HIDDEN, HIDDEN), lambda i: (0, 0)),
            pl.BlockSpec((1, HIDDEN), lambda i: (0, 0)),
        ],
        out_specs=pl.BlockSpec((ROW_BLOCK, HIDDEN), lambda i: (i, 0)),
        out_shape=jax.ShapeDtypeStruct((N_NODES, HIDDEN), jnp.float32),
    )(p0, p1, W, b2d)


def kernel(t, x, edge_index, A_values, W, b):
    dst = edge_index[0]
    src = edge_index[1]
    zeros = jnp.zeros((ROWS_PER_TILE, HIDDEN), jnp.float32)
    partials = _segment_sum_sc(x, src, dst, A_values, zeros)
    return _linear_relu(partials[0], partials[1], W, b.reshape(1, HIDDEN))


# speedup vs baseline: 4.6883x; 4.6883x over previous
import functools

import jax
import jax.numpy as jnp
from jax import lax
from jax.experimental import pallas as pl
from jax.experimental.pallas import tpu as pltpu
from jax.experimental.pallas import tpu_sc as plsc

N_NODES = 10000
N_EDGES = 320000
HIDDEN = 128
LANES = 16

CHUNK = 128
NCHUNKS = N_EDGES // CHUNK
NTILES = 32
ITERS = -(-NCHUNKS // NTILES)
ROWS_PER_TILE = 640
N_PAD = 16 * ROWS_PER_TILE

_mesh = plsc.VectorSubcoreMesh(core_axis_name="c", subcore_axis_name="s")


@functools.partial(
    pl.kernel,
    out_type=jax.ShapeDtypeStruct((2, N_PAD, HIDDEN), jnp.float32),
    mesh=_mesh,
    scratch_types=[
        pltpu.VMEM((CHUNK,), jnp.int32),
        pltpu.VMEM((CHUNK,), jnp.int32),
        pltpu.VMEM((CHUNK,), jnp.float32),
        pltpu.VMEM((CHUNK, HIDDEN), jnp.float32),
        pltpu.VMEM_SHARED((N_PAD, HIDDEN), jnp.float32),
        pltpu.SemaphoreType.DMA,
    ],
    compiler_params=pltpu.CompilerParams(needs_layout_passes=False),
)
def _segment_sum_sc(x_hbm, src_hbm, dst_hbm, a_hbm, zeros_hbm, out_hbm,
                    src_v, dst_v, a_v, rows_v, agg_sh, sem):
    cid = lax.axis_index("c")
    sid = lax.axis_index("s")
    wid = sid * 2 + cid
    row0 = sid * ROWS_PER_TILE

    pltpu.sync_copy(zeros_hbm, agg_sh.at[pl.ds(row0, ROWS_PER_TILE)])
    plsc.subcore_barrier()

    def chunk_body(i, carry):
        c = wid + NTILES * i

        @pl.when(c < NCHUNKS)
        def _():
            base = c * CHUNK
            pltpu.sync_copy(src_hbm.at[pl.ds(base, CHUNK)], src_v)
            pltpu.sync_copy(dst_hbm.at[pl.ds(base, CHUNK)], dst_v)
            pltpu.sync_copy(a_hbm.at[pl.ds(base, CHUNK)], a_v)
            pltpu.async_copy(x_hbm.at[src_v], rows_v, sem).wait()

            def edge_body(e, carry2):
                a16 = plsc.load_gather(a_v, [jnp.full((LANES,), e, jnp.int32)])
                for j in range(HIDDEN // LANES):
                    sl = pl.ds(j * LANES, LANES)
                    rows_v[e, sl] = rows_v[e, sl] * a16
                return carry2

            lax.fori_loop(0, CHUNK, edge_body, 0)
            pltpu.sync_copy(rows_v, agg_sh.at[dst_v], add=True)

        return carry

    lax.fori_loop(0, ITERS, chunk_body, 0)

    plsc.subcore_barrier()
    pltpu.sync_copy(agg_sh.at[pl.ds(row0, ROWS_PER_TILE)],
                    out_hbm.at[cid, pl.ds(row0, ROWS_PER_TILE)])


ROW_BLOCK = 1000


def _linear_relu_body(p0_ref, p1_ref, w_ref, b_ref, o_ref):
    s = p0_ref[...] + p1_ref[...]
    y = lax.dot_general(s, w_ref[...], (((1,), (1,)), ((), ())),
                        preferred_element_type=jnp.float32)
    o_ref[...] = jnp.maximum(y + b_ref[...], 0.0)


def _linear_relu(p0, p1, W, b2d):
    return pl.pallas_call(
        _linear_relu_body,
        grid=(N_NODES // ROW_BLOCK,),
        in_specs=[
            pl.BlockSpec((ROW_BLOCK, HIDDEN), lambda i: (i, 0)),
            pl.BlockSpec((ROW_BLOCK, HIDDEN), lambda i: (i, 0)),
            pl.BlockSpec((HIDDEN, HIDDEN), lambda i: (0, 0)),
            pl.BlockSpec((1, HIDDEN), lambda i: (0, 0)),
        ],
        out_specs=pl.BlockSpec((ROW_BLOCK, HIDDEN), lambda i: (i, 0)),
        out_shape=jax.ShapeDtypeStruct((N_NODES, HIDDEN), jnp.float32),
    )(p0, p1, W, b2d)


def kernel(t, x, edge_index, A_values, W, b):
    dst = edge_index[0]
    src = edge_index[1]
    zeros = jnp.zeros((ROWS_PER_TILE, HIDDEN), jnp.float32)
    partials = _segment_sum_sc(x, src, dst, A_values, zeros)
    return _linear_relu(partials[0, :N_NODES], partials[1, :N_NODES],
                        W, b.reshape(1, HIDDEN))


# speedup vs baseline: 9.0266x; 1.9253x over previous
import functools

import jax
import jax.numpy as jnp
from jax import lax
from jax.experimental import pallas as pl
from jax.experimental.pallas import tpu as pltpu
from jax.experimental.pallas import tpu_sc as plsc

N_NODES = 10000
N_EDGES = 320000
HIDDEN = 128
LANES = 16

NTILES = 32
CHUNK = 128
CHUNKS_PER_TILE = 80
PER_PHASE = 40
E_PAD = NTILES * CHUNKS_PER_TILE * CHUNK
NCHUNKS = E_PAD // CHUNK
ROWS_PER_TILE = 640
N_PAD = 16 * ROWS_PER_TILE

_mesh = plsc.VectorSubcoreMesh(core_axis_name="c", subcore_axis_name="s")


@functools.partial(
    pl.kernel,
    out_type=jax.ShapeDtypeStruct((2, N_PAD, HIDDEN), jnp.float32),
    mesh=_mesh,
    scratch_types=[
        pltpu.VMEM((PER_PHASE, CHUNK), jnp.int32),
        pltpu.VMEM((PER_PHASE, CHUNK), jnp.int32),
        pltpu.VMEM((PER_PHASE, CHUNK), jnp.float32),
        pltpu.VMEM((CHUNK, HIDDEN), jnp.float32),
        pltpu.VMEM((CHUNK, HIDDEN), jnp.float32),
        pltpu.VMEM_SHARED((N_PAD, HIDDEN), jnp.float32),
        pltpu.SemaphoreType.DMA,
        pltpu.SemaphoreType.DMA,
        pltpu.SemaphoreType.DMA,
        pltpu.SemaphoreType.DMA,
    ],
    compiler_params=pltpu.CompilerParams(needs_layout_passes=False),
)
def _segment_sum_sc(x_hbm, src_hbm, dst_hbm, a_hbm, zeros_hbm, out_hbm,
                    src_all, dst_all, a_all, rows0, rows1, agg_sh,
                    gsem0, gsem1, ssem0, ssem1):
    cid = lax.axis_index("c")
    sid = lax.axis_index("s")
    wid = sid * 2 + cid
    row0 = sid * ROWS_PER_TILE
    chunk0 = wid * CHUNKS_PER_TILE

    pltpu.sync_copy(zeros_hbm, agg_sh.at[pl.ds(row0, ROWS_PER_TILE)])
    plsc.subcore_barrier()

    def scale_rows(rows, i):
        ii = jnp.full((LANES,), i, jnp.int32)

        def group_body(g, carry):
            for l in range(LANES):
                e = g * LANES + l
                aw = plsc.load_gather(a_all, [ii, jnp.full((LANES,), e, jnp.int32)])
                for j in range(HIDDEN // LANES):
                    sl = pl.ds(j * LANES, LANES)
                    rows[e, sl] = rows[e, sl] * aw
            return carry

        lax.fori_loop(0, CHUNK // LANES, group_body, 0)

    def step(i, own, other, gsem_own, gsem_other, ssem_other):
        @pl.when(i >= 1)
        def _():
            pltpu.make_async_copy(
                other, agg_sh.at[dst_all.at[i - 1]], ssem_other).wait()

        @pl.when(i + 1 < PER_PHASE)
        def _():
            pltpu.async_copy(x_hbm.at[src_all.at[i + 1]], other, gsem_other)

        pltpu.make_async_copy(x_hbm.at[src_all.at[i]], own, gsem_own).wait()
        scale_rows(own, i)

    def chunk_body(i, carry):
        even = lax.rem(i, 2) == 0

        @pl.when(even)
        def _():
            step(i, rows0, rows1, gsem0, gsem1, ssem1)
            pltpu.async_copy(rows0, agg_sh.at[dst_all.at[i]], ssem0, add=True)

        @pl.when(jnp.logical_not(even))
        def _():
            step(i, rows1, rows0, gsem1, gsem0, ssem0)
            pltpu.async_copy(rows1, agg_sh.at[dst_all.at[i]], ssem1, add=True)

        return carry

    for phase in range(CHUNKS_PER_TILE // PER_PHASE):
        base = chunk0 + phase * PER_PHASE
        pltpu.sync_copy(src_hbm.at[pl.ds(base, PER_PHASE)], src_all)
        pltpu.sync_copy(dst_hbm.at[pl.ds(base, PER_PHASE)], dst_all)
        pltpu.sync_copy(a_hbm.at[pl.ds(base, PER_PHASE)], a_all)
        pltpu.async_copy(x_hbm.at[src_all.at[0]], rows0, gsem0)
        lax.fori_loop(0, PER_PHASE, chunk_body, 0)
        pltpu.make_async_copy(
            rows1, agg_sh.at[dst_all.at[PER_PHASE - 1]], ssem1).wait()

    plsc.subcore_barrier()
    pltpu.sync_copy(agg_sh.at[pl.ds(row0, ROWS_PER_TILE)],
                    out_hbm.at[cid, pl.ds(row0, ROWS_PER_TILE)])


ROW_BLOCK = 1000


def _linear_relu_body(p0_ref, p1_ref, w_ref, b_ref, o_ref):
    s = p0_ref[...] + p1_ref[...]
    y = lax.dot_general(s, w_ref[...], (((1,), (1,)), ((), ())),
                        preferred_element_type=jnp.float32)
    o_ref[...] = jnp.maximum(y + b_ref[...], 0.0)


def _linear_relu(p0, p1, W, b2d):
    return pl.pallas_call(
        _linear_relu_body,
        grid=(N_NODES // ROW_BLOCK,),
        in_specs=[
            pl.BlockSpec((ROW_BLOCK, HIDDEN), lambda i: (i, 0)),
            pl.BlockSpec((ROW_BLOCK, HIDDEN), lambda i: (i, 0)),
            pl.BlockSpec((HIDDEN, HIDDEN), lambda i: (0, 0)),
            pl.BlockSpec((1, HIDDEN), lambda i: (0, 0)),
        ],
        out_specs=pl.BlockSpec((ROW_BLOCK, HIDDEN), lambda i: (i, 0)),
        out_shape=jax.ShapeDtypeStruct((N_NODES, HIDDEN), jnp.float32),
    )(p0, p1, W, b2d)


def kernel(t, x, edge_index, A_values, W, b):
    dst = edge_index[0]
    src = edge_index[1]
    npad = E_PAD - N_EDGES
    pad_lanes = jnp.arange(npad, dtype=jnp.int32)
    src_p = jnp.concatenate([src, pad_lanes % N_NODES]).reshape(NCHUNKS, CHUNK)
    dst_p = jnp.concatenate(
        [dst, N_NODES + (pad_lanes % (N_PAD - N_NODES))]).reshape(NCHUNKS, CHUNK)
    a_p = jnp.concatenate(
        [A_values, jnp.zeros((npad,), jnp.float32)]).reshape(NCHUNKS, CHUNK)
    zeros = jnp.zeros((ROWS_PER_TILE, HIDDEN), jnp.float32)
    partials = _segment_sum_sc(x, src_p, dst_p, a_p, zeros)
    return _linear_relu(partials[0, :N_NODES], partials[1, :N_NODES],
                        W, b.reshape(1, HIDDEN))
